# Initial kernel scaffold; baseline (speedup 1.0000x reference)
import jax
import jax.numpy as jnp
from jax.experimental import pallas as pl


def kernel(x, rel_x, rel_edge_attr, fc1, nn_W, nn_b, conv1_b, conv2_b, fc2_W, fc2_b, edge_index, edge_type, rel_edge_index):
    raise NotImplementedError("write your pallas kernel here")


# speedup vs baseline: 10.3055x; 10.3055x over previous
import functools

import jax
import jax.numpy as jnp
from jax import lax
from jax.experimental import pallas as pl
from jax.experimental.pallas import tpu as pltpu
from jax.experimental.pallas import tpu_sc as plsc

N_NODES = 100000
NUM_FEATURES = 128
DIM = 16
NUM_CLASSES = 16
N_REL = 8
N_REL_EDGES = 64

NC, NS = 2, 16
E_PAD = 1638400
N_GROUPS = E_PAD // 128
GROUPS_PER_TILE = N_GROUPS // (NC * NS)
GROUPS_PER_CHUNK = 8
CHUNKS = GROUPS_PER_TILE // GROUPS_PER_CHUNK
N_PAD = 102400
TILE_ROWS = N_PAD // NS


def _rel_body(rel_x_ref, attr_ref, src_ref, dst_ref, nnw_ref, nnb_ref, out_ref):
    rel_x = rel_x_ref[...]
    attr = attr_ref[...]
    iota = lax.broadcasted_iota(jnp.int32, (N_REL_EDGES, N_REL), 1)
    oh_src = (src_ref[...] == iota).astype(jnp.float32)
    oh_dst = (dst_ref[...] == iota).astype(jnp.float32)
    ones = jnp.ones((N_REL_EDGES, 1), jnp.float32)
    cnt = lax.dot_general(oh_dst, ones, (((0,), (0,)), ((), ())))
    inv = 1.0 / jnp.maximum(cnt, 1.0)

    def conv(h):
        msgs = jnp.dot(oh_src, h) * attr
        summed = lax.dot_general(oh_dst, msgs, (((0,), (0,)), ((), ())))
        return jax.nn.relu(summed * inv + h)

    r = conv(conv(rel_x))
    out_ref[...] = jnp.dot(r, nnw_ref[...]) + nnb_ref[...]


def _rel_call(rel_x, rel_edge_attr, rel_edge_index, nn_W, nn_b):
    return pl.pallas_call(
        _rel_body,
        out_shape=jax.ShapeDtypeStruct((N_REL, DIM), jnp.float32),
    )(rel_x, rel_edge_attr[:, None],
      rel_edge_index[0][:, None], rel_edge_index[1][:, None],
      nn_W, nn_b[None, :])


def _gidx_body(src_ref, typ_ref, out_ref):
    out_ref[...] = typ_ref[...] * N_PAD + src_ref[...]


def _gidx_call(src2d, typ2d):
    blk = (1600, 128)
    return pl.pallas_call(
        _gidx_body,
        out_shape=jax.ShapeDtypeStruct((N_GROUPS, 128), jnp.int32),
        grid=(N_GROUPS // blk[0],),
        in_specs=[pl.BlockSpec(blk, lambda i: (i, 0)),
                  pl.BlockSpec(blk, lambda i: (i, 0))],
        out_specs=pl.BlockSpec(blk, lambda i: (i, 0)),
    )(src2d, typ2d)


def _embed_body(x_ref, fc1_ref, w_ref, out_ref):
    h0 = jnp.dot(x_ref[...], fc1_ref[...],
                 preferred_element_type=jnp.float32)
    w = w_ref[...]
    for t in range(N_REL):
        out_ref[t] = h0 * w[t][None, :]


def _embed_call(x, fc1, w_rel):
    B = 1000
    return pl.pallas_call(
        _embed_body,
        out_shape=jax.ShapeDtypeStruct((N_REL, N_PAD, DIM), jnp.float32),
        grid=(N_NODES // B,),
        in_specs=[pl.BlockSpec((B, NUM_FEATURES), lambda i: (i, 0)),
                  pl.BlockSpec((NUM_FEATURES, DIM), lambda i: (0, 0)),
                  pl.BlockSpec((N_REL, DIM), lambda i: (0, 0))],
        out_specs=pl.BlockSpec((N_REL, B, DIM), lambda i: (0, i, 0)),
    )(x, fc1, w_rel)


def _make_edge_pass(with_counts):
    mesh = plsc.VectorSubcoreMesh(
        core_axis_name="c", subcore_axis_name="s",
        num_cores=NC, num_subcores=NS)
    out_type = [jax.ShapeDtypeStruct((NC, N_PAD, DIM), jnp.float32)]
    if with_counts:
        out_type.append(jax.ShapeDtypeStruct((NC, 1, N_PAD), jnp.float32))
    scratch = [
        pltpu.VMEM_SHARED((N_PAD, DIM), jnp.float32),
        pltpu.VMEM_SHARED((N_PAD,), jnp.float32),
        pltpu.VMEM((GROUPS_PER_CHUNK, 128), jnp.int32),
        pltpu.VMEM((GROUPS_PER_CHUNK, 128), jnp.int32),
        pltpu.VMEM((GROUPS_PER_CHUNK * 128, DIM), jnp.float32),
        pltpu.VMEM((128,), jnp.float32),
        pltpu.SemaphoreType.DMA,
        pltpu.SemaphoreType.DMA,
        pltpu.SemaphoreType.DMA,
    ]

    def body(hall_ref, g_ref, d_ref, z2_ref, z1_ref, *rest):
        if with_counts:
            acc_out, cnt_out = rest[0], rest[1]
            rest = rest[2:]
        else:
            acc_out = rest[0]
            cnt_out = None
            rest = rest[1:]
        acc_sh, cnt_sh, gbuf, dbuf, rows, ones_v, gsem, ssem, csem = rest

        c = lax.axis_index("c")
        s = lax.axis_index("s")
        off = s * TILE_ROWS

        pltpu.sync_copy(z2_ref, acc_sh.at[pl.ds(off, TILE_ROWS)])
        if with_counts:
            pltpu.sync_copy(z1_ref, cnt_sh.at[pl.ds(off, TILE_ROWS)])
            for k in range(8):
                ones_v[pl.ds(k * 16, 16)] = jnp.full((16,), 1.0, jnp.float32)
        plsc.subcore_barrier()

        g0 = (c * NS + s) * GROUPS_PER_TILE

        def chunk(i, carry):
            grp = g0 + i * GROUPS_PER_CHUNK
            pltpu.sync_copy(g_ref.at[pl.ds(grp, GROUPS_PER_CHUNK)], gbuf)
            pltpu.sync_copy(d_ref.at[pl.ds(grp, GROUPS_PER_CHUNK)], dbuf)
            gcs = [pltpu.async_copy(hall_ref.at[gbuf.at[j]],
                                    rows.at[pl.ds(j * 128, 128)], gsem)
                   for j in range(GROUPS_PER_CHUNK)]
            for cp in gcs:
                cp.wait()
            scs = [pltpu.async_copy(rows.at[pl.ds(j * 128, 128)],
                                    acc_sh.at[dbuf.at[j]], ssem, add=True)
                   for j in range(GROUPS_PER_CHUNK)]
            ccs = []
            if with_counts:
                ccs = [pltpu.async_copy(ones_v, cnt_sh.at[dbuf.at[j]], csem,
                                        add=True)
                       for j in range(GROUPS_PER_CHUNK)]
            for cp in scs:
                cp.wait()
            for cp in ccs:
                cp.wait()
            return carry

        lax.fori_loop(0, CHUNKS, chunk, 0)
        plsc.subcore_barrier()

        pltpu.sync_copy(acc_sh.at[pl.ds(off, TILE_ROWS)],
                        acc_out.at[c, pl.ds(off, TILE_ROWS)])
        if with_counts:
            pltpu.sync_copy(cnt_sh.at[pl.ds(off, TILE_ROWS)],
                            cnt_out.at[c, 0, pl.ds(off, TILE_ROWS)])

    return pl.kernel(body, out_type=out_type, mesh=mesh, scratch_types=scratch,
                     compiler_params=pltpu.CompilerParams(
                         use_tc_tiling_on_sc=False))


_edge_pass_counts = _make_edge_pass(True)
_edge_pass_plain = _make_edge_pass(False)


def _epi1_body(acc_ref, cnt_ref, b_ref, w_ref, out_ref):
    i = pl.program_id(0)
    B = acc_ref.shape[1]
    a = acc_ref[0] + acc_ref[1]
    cnt = jnp.maximum(cnt_ref[0, 0, pl.ds(i * B, B)]
                      + cnt_ref[1, 0, pl.ds(i * B, B)], 1.0)
    h1 = jnp.tanh(a / cnt[:, None] + b_ref[...])
    w = w_ref[...]
    for t in range(N_REL):
        out_ref[t] = h1 * w[t][None, :]


def _epi1_call(acc, cnt3, conv1_b, w_rel):
    B = 1024
    return pl.pallas_call(
        _epi1_body,
        out_shape=jax.ShapeDtypeStruct((N_REL, N_PAD, DIM), jnp.float32),
        grid=(N_PAD // B,),
        in_specs=[pl.BlockSpec((NC, B, DIM), lambda i: (0, i, 0)),
                  pl.BlockSpec((NC, 1, N_PAD), lambda i: (0, 0, 0)),
                  pl.BlockSpec((1, DIM), lambda i: (0, 0)),
                  pl.BlockSpec((N_REL, DIM), lambda i: (0, 0))],
        out_specs=pl.BlockSpec((N_REL, B, DIM), lambda i: (0, i, 0)),
    )(acc, cnt3, conv1_b[None, :], w_rel)


def _final_body(acc_ref, cnt_ref, b_ref, w2_ref, b2_ref, out_ref):
    i = pl.program_id(0)
    B = acc_ref.shape[1]
    a = acc_ref[0] + acc_ref[1]
    cnt = jnp.maximum(cnt_ref[0, 0, pl.ds(i * B, B)]
                      + cnt_ref[1, 0, pl.ds(i * B, B)], 1.0)
    h2 = jnp.tanh(a / cnt[:, None] + b_ref[...])
    out_ref[...] = jnp.dot(h2, w2_ref[...],
                           preferred_element_type=jnp.float32) + b2_ref[...]


def _final_call(acc, cnt3, conv2_b, fc2_W, fc2_b):
    B = 1024
    return pl.pallas_call(
        _final_body,
        out_shape=jax.ShapeDtypeStruct((N_PAD, NUM_CLASSES), jnp.float32),
        grid=(N_PAD // B,),
        in_specs=[pl.BlockSpec((NC, B, DIM), lambda i: (0, i, 0)),
                  pl.BlockSpec((NC, 1, N_PAD), lambda i: (0, 0, 0)),
                  pl.BlockSpec((1, DIM), lambda i: (0, 0)),
                  pl.BlockSpec((DIM, NUM_CLASSES), lambda i: (0, 0)),
                  pl.BlockSpec((1, NUM_CLASSES), lambda i: (0, 0))],
        out_specs=pl.BlockSpec((B, NUM_CLASSES), lambda i: (i, 0)),
    )(acc, cnt3, conv2_b[None, :], fc2_W, fc2_b[None, :])


def kernel(x, rel_x, rel_edge_attr, fc1, nn_W, nn_b, conv1_b, conv2_b,
           fc2_W, fc2_b, edge_index, edge_type, rel_edge_index):
    E = edge_index.shape[1]
    pad = E_PAD - E
    src = jnp.concatenate([edge_index[0], jnp.zeros((pad,), jnp.int32)])
    dst = jnp.concatenate([edge_index[1],
                           jnp.full((pad,), N_NODES, jnp.int32)])
    typ = jnp.concatenate([edge_type, jnp.zeros((pad,), jnp.int32)])
    src2d = src.reshape(N_GROUPS, 128)
    dst2d = dst.reshape(N_GROUPS, 128)
    typ2d = typ.reshape(N_GROUPS, 128)

    z2 = jnp.zeros((TILE_ROWS, DIM), jnp.float32)
    z1 = jnp.zeros((TILE_ROWS,), jnp.float32)

    w_rel = _rel_call(rel_x, rel_edge_attr, rel_edge_index, nn_W, nn_b)
    g2d = _gidx_call(src2d, typ2d)
    hall1 = _embed_call(x, fc1, w_rel).reshape(N_REL * N_PAD, DIM)

    acc1, cnt3 = _edge_pass_counts(hall1, g2d, dst2d, z2, z1)
    hall2 = _epi1_call(acc1, cnt3, conv1_b, w_rel).reshape(N_REL * N_PAD, DIM)

    acc2 = _edge_pass_plain(hall2, g2d, dst2d, z2, z1)
    if isinstance(acc2, (list, tuple)):
        acc2 = acc2[0]
    return _final_call(acc2, cnt3, conv2_b, fc2_W, fc2_b)[:N_NODES]


# speedup vs baseline: 11.4482x; 1.1109x over previous
import functools

import jax
import jax.numpy as jnp
from jax import lax
from jax.experimental import pallas as pl
from jax.experimental.pallas import tpu as pltpu
from jax.experimental.pallas import tpu_sc as plsc

N_NODES = 100000
NUM_FEATURES = 128
DIM = 16
NUM_CLASSES = 16
N_REL = 8
N_REL_EDGES = 64

NC, NS = 2, 16
E_PAD = 1638400
N_GROUPS = E_PAD // 128
GROUPS_PER_TILE = N_GROUPS // (NC * NS)
GROUPS_PER_CHUNK = 8
CHUNKS = GROUPS_PER_TILE // GROUPS_PER_CHUNK
N_PAD = 102400
TILE_ROWS = N_PAD // NS


def _rel_body(rel_x_ref, attr_ref, src_ref, dst_ref, nnw_ref, nnb_ref, out_ref):
    rel_x = rel_x_ref[...]
    attr = attr_ref[...]
    iota = lax.broadcasted_iota(jnp.int32, (N_REL_EDGES, N_REL), 1)
    oh_src = (src_ref[...] == iota).astype(jnp.float32)
    oh_dst = (dst_ref[...] == iota).astype(jnp.float32)
    ones = jnp.ones((N_REL_EDGES, 1), jnp.float32)
    cnt = lax.dot_general(oh_dst, ones, (((0,), (0,)), ((), ())))
    inv = 1.0 / jnp.maximum(cnt, 1.0)

    def conv(h):
        msgs = jnp.dot(oh_src, h) * attr
        summed = lax.dot_general(oh_dst, msgs, (((0,), (0,)), ((), ())))
        return jax.nn.relu(summed * inv + h)

    r = conv(conv(rel_x))
    out_ref[...] = jnp.dot(r, nnw_ref[...]) + nnb_ref[...]


def _rel_call(rel_x, rel_edge_attr, rel_edge_index, nn_W, nn_b):
    return pl.pallas_call(
        _rel_body,
        out_shape=jax.ShapeDtypeStruct((N_REL, DIM), jnp.float32),
    )(rel_x, rel_edge_attr[:, None],
      rel_edge_index[0][:, None], rel_edge_index[1][:, None],
      nn_W, nn_b[None, :])


def _gidx_body(src_ref, typ_ref, out_ref):
    out_ref[...] = typ_ref[...] * N_PAD + src_ref[...]


def _gidx_call(src2d, typ2d):
    blk = (1600, 128)
    return pl.pallas_call(
        _gidx_body,
        out_shape=jax.ShapeDtypeStruct((N_GROUPS, 128), jnp.int32),
        grid=(N_GROUPS // blk[0],),
        in_specs=[pl.BlockSpec(blk, lambda i: (i, 0)),
                  pl.BlockSpec(blk, lambda i: (i, 0))],
        out_specs=pl.BlockSpec(blk, lambda i: (i, 0)),
    )(src2d, typ2d)


def _embed_body(x_ref, fc1_ref, w_ref, out_ref):
    h0 = jnp.dot(x_ref[...], fc1_ref[...],
                 preferred_element_type=jnp.float32)
    w = w_ref[...]
    for t in range(N_REL):
        out_ref[t] = h0 * w[t][None, :]


def _embed_call(x, fc1, w_rel):
    B = 1000
    return pl.pallas_call(
        _embed_body,
        out_shape=jax.ShapeDtypeStruct((N_REL, N_PAD, DIM), jnp.float32),
        grid=(N_NODES // B,),
        in_specs=[pl.BlockSpec((B, NUM_FEATURES), lambda i: (i, 0)),
                  pl.BlockSpec((NUM_FEATURES, DIM), lambda i: (0, 0)),
                  pl.BlockSpec((N_REL, DIM), lambda i: (0, 0))],
        out_specs=pl.BlockSpec((N_REL, B, DIM), lambda i: (0, i, 0)),
    )(x, fc1, w_rel)


EDGES_PER_TILE = E_PAD // (NC * NS)
EPC = 512
SC_CHUNKS = EDGES_PER_TILE // EPC


def _make_edge_pass(with_counts):
    mesh = plsc.VectorSubcoreMesh(
        core_axis_name="c", subcore_axis_name="s",
        num_cores=NC, num_subcores=NS)
    out_type = [jax.ShapeDtypeStruct((NC, N_PAD, DIM), jnp.float32)]
    if with_counts:
        out_type.append(jax.ShapeDtypeStruct((NC, 1, N_PAD), jnp.float32))
    scratch = [
        pltpu.VMEM_SHARED((N_PAD, DIM), jnp.float32),
        pltpu.VMEM_SHARED((N_PAD,), jnp.float32),
        pltpu.VMEM((EPC,), jnp.int32),
        pltpu.VMEM((EPC,), jnp.int32),
        pltpu.VMEM((EPC,), jnp.int32),
        pltpu.VMEM((EPC,), jnp.int32),
        pltpu.VMEM((EPC,), jnp.int32),
        pltpu.VMEM((EPC,), jnp.int32),
        pltpu.VMEM((EPC, DIM), jnp.float32),
        pltpu.VMEM((EPC, DIM), jnp.float32),
        pltpu.VMEM((EPC,), jnp.float32),
    ] + [pltpu.SemaphoreType.DMA] * 8

    def body(hall_ref, g_ref, d_ref, z2_ref, z1_ref, *rest):
        if with_counts:
            acc_out, cnt_out = rest[0], rest[1]
            rest = rest[2:]
        else:
            acc_out = rest[0]
            cnt_out = None
            rest = rest[1:]
        (acc_sh, cnt_sh, gb0, gb1, db0, db1, db2, db3, rw0, rw1, ones_v,
         isem0, isem1, gsem0, gsem1, ssem0, ssem1, csem0, csem1) = rest
        gbufs, dbufs, rows = [gb0, gb1], [db0, db1, db2, db3], [rw0, rw1]
        isems, gsems, ssems, csems = ([isem0, isem1], [gsem0, gsem1],
                                      [ssem0, ssem1], [csem0, csem1])

        cid = lax.axis_index("c")
        sid = lax.axis_index("s")
        off = sid * TILE_ROWS

        pltpu.sync_copy(z2_ref, acc_sh.at[pl.ds(off, TILE_ROWS)])
        if with_counts:
            pltpu.sync_copy(z1_ref, cnt_sh.at[pl.ds(off, TILE_ROWS)])
            for k in range(EPC // 16):
                ones_v[pl.ds(k * 16, 16)] = jnp.full((16,), 1.0, jnp.float32)
        plsc.subcore_barrier()

        ebase = (cid * NS + sid) * EDGES_PER_TILE

        def fire_idx(chunk, slot, dslot):
            base = ebase + chunk * EPC
            pltpu.async_copy(g_ref.at[pl.ds(base, EPC)], gbufs[slot],
                             isems[slot])
            pltpu.async_copy(d_ref.at[pl.ds(base, EPC)], dbufs[dslot],
                             isems[slot])

        def drain_idx(slot, dslot):
            pltpu.make_async_copy(g_ref.at[pl.ds(ebase, EPC)], gbufs[slot],
                                  isems[slot]).wait()
            pltpu.make_async_copy(d_ref.at[pl.ds(ebase, EPC)], dbufs[dslot],
                                  isems[slot]).wait()

        def fire_scat(slot, dslot):
            pltpu.async_copy(rows[slot], acc_sh.at[dbufs[dslot]], ssems[slot],
                             add=True)
            if with_counts:
                pltpu.async_copy(ones_v, cnt_sh.at[dbufs[dslot]], csems[slot],
                                 add=True)

        def drain_scat(slot, dslot):
            pltpu.make_async_copy(rows[slot], acc_sh.at[dbufs[dslot]],
                                  ssems[slot]).wait()
            if with_counts:
                pltpu.make_async_copy(ones_v, cnt_sh.at[dbufs[dslot]],
                                      csems[slot]).wait()

        def drain_gather(slot):
            pltpu.make_async_copy(hall_ref.at[gbufs[slot]], rows[slot],
                                  gsems[slot]).wait()

        fire_idx(0, 0, 0)

        def quad(k, carry):
            for q in range(4):
                s = q % 2
                j = 4 * k + q

                def step_drain_scat(s=s, q=q):
                    drain_scat(s, (q + 2) % 4)
                if q < 2:
                    pl.when(k > 0)(step_drain_scat)
                else:
                    step_drain_scat()

                drain_idx(s, q)
                pltpu.async_copy(hall_ref.at[gbufs[s]], rows[s], gsems[s])

                def step_tail(s=s, q=q):
                    drain_gather(1 - s)
                    fire_scat(1 - s, (q + 3) % 4)
                if q == 0:
                    pl.when(k > 0)(step_tail)
                else:
                    step_tail()
                fire_idx(jnp.minimum(j + 1, SC_CHUNKS - 1), 1 - s,
                         (q + 1) % 4)
            return carry

        lax.fori_loop(0, SC_CHUNKS // 4, quad, 0)

        drain_gather(1)
        fire_scat(1, 3)
        drain_scat(0, 2)
        drain_scat(1, 3)
        drain_idx(0, 0)
        plsc.subcore_barrier()

        pltpu.sync_copy(acc_sh.at[pl.ds(off, TILE_ROWS)],
                        acc_out.at[cid, pl.ds(off, TILE_ROWS)])
        if with_counts:
            pltpu.sync_copy(cnt_sh.at[pl.ds(off, TILE_ROWS)],
                            cnt_out.at[cid, 0, pl.ds(off, TILE_ROWS)])

    return pl.kernel(body, out_type=out_type, mesh=mesh, scratch_types=scratch,
                     compiler_params=pltpu.CompilerParams(
                         use_tc_tiling_on_sc=False))


_edge_pass_counts = _make_edge_pass(True)
_edge_pass_plain = _make_edge_pass(False)


def _epi1_body(acc_ref, cnt_ref, b_ref, w_ref, out_ref):
    i = pl.program_id(0)
    B = acc_ref.shape[1]
    a = acc_ref[0] + acc_ref[1]
    cnt = jnp.maximum(cnt_ref[0, 0, pl.ds(i * B, B)]
                      + cnt_ref[1, 0, pl.ds(i * B, B)], 1.0)
    h1 = jnp.tanh(a / cnt[:, None] + b_ref[...])
    w = w_ref[...]
    for t in range(N_REL):
        out_ref[t] = h1 * w[t][None, :]


def _epi1_call(acc, cnt3, conv1_b, w_rel):
    B = 1024
    return pl.pallas_call(
        _epi1_body,
        out_shape=jax.ShapeDtypeStruct((N_REL, N_PAD, DIM), jnp.float32),
        grid=(N_PAD // B,),
        in_specs=[pl.BlockSpec((NC, B, DIM), lambda i: (0, i, 0)),
                  pl.BlockSpec((NC, 1, N_PAD), lambda i: (0, 0, 0)),
                  pl.BlockSpec((1, DIM), lambda i: (0, 0)),
                  pl.BlockSpec((N_REL, DIM), lambda i: (0, 0))],
        out_specs=pl.BlockSpec((N_REL, B, DIM), lambda i: (0, i, 0)),
    )(acc, cnt3, conv1_b[None, :], w_rel)


def _final_body(acc_ref, cnt_ref, b_ref, w2_ref, b2_ref, out_ref):
    i = pl.program_id(0)
    B = acc_ref.shape[1]
    a = acc_ref[0] + acc_ref[1]
    cnt = jnp.maximum(cnt_ref[0, 0, pl.ds(i * B, B)]
                      + cnt_ref[1, 0, pl.ds(i * B, B)], 1.0)
    h2 = jnp.tanh(a / cnt[:, None] + b_ref[...])
    out_ref[...] = jnp.dot(h2, w2_ref[...],
                           preferred_element_type=jnp.float32) + b2_ref[...]


def _final_call(acc, cnt3, conv2_b, fc2_W, fc2_b):
    B = 1024
    return pl.pallas_call(
        _final_body,
        out_shape=jax.ShapeDtypeStruct((N_PAD, NUM_CLASSES), jnp.float32),
        grid=(N_PAD // B,),
        in_specs=[pl.BlockSpec((NC, B, DIM), lambda i: (0, i, 0)),
                  pl.BlockSpec((NC, 1, N_PAD), lambda i: (0, 0, 0)),
                  pl.BlockSpec((1, DIM), lambda i: (0, 0)),
                  pl.BlockSpec((DIM, NUM_CLASSES), lambda i: (0, 0)),
                  pl.BlockSpec((1, NUM_CLASSES), lambda i: (0, 0))],
        out_specs=pl.BlockSpec((B, NUM_CLASSES), lambda i: (i, 0)),
    )(acc, cnt3, conv2_b[None, :], fc2_W, fc2_b[None, :])


def kernel(x, rel_x, rel_edge_attr, fc1, nn_W, nn_b, conv1_b, conv2_b,
           fc2_W, fc2_b, edge_index, edge_type, rel_edge_index):
    E = edge_index.shape[1]
    pad = E_PAD - E
    src = jnp.concatenate([edge_index[0], jnp.zeros((pad,), jnp.int32)])
    dst = jnp.concatenate([edge_index[1],
                           jnp.full((pad,), N_NODES, jnp.int32)])
    typ = jnp.concatenate([edge_type, jnp.zeros((pad,), jnp.int32)])
    src2d = src.reshape(N_GROUPS, 128)
    typ2d = typ.reshape(N_GROUPS, 128)

    z2 = jnp.zeros((TILE_ROWS, DIM), jnp.float32)
    z1 = jnp.zeros((TILE_ROWS,), jnp.float32)

    w_rel = _rel_call(rel_x, rel_edge_attr, rel_edge_index, nn_W, nn_b)
    g2d = _gidx_call(src2d, typ2d)
    hall1 = _embed_call(x, fc1, w_rel).reshape(N_REL * N_PAD, DIM)

    acc1, cnt3 = _edge_pass_counts(hall1, g2d.reshape(E_PAD), dst, z2, z1)
    hall2 = _epi1_call(acc1, cnt3, conv1_b, w_rel).reshape(N_REL * N_PAD, DIM)

    acc2 = _edge_pass_plain(hall2, g2d.reshape(E_PAD), dst, z2, z1)
    if isinstance(acc2, (list, tuple)):
        acc2 = acc2[0]
    return _final_call(acc2, cnt3, conv2_b, fc2_W, fc2_b)[:N_NODES]


# speedup vs baseline: 16.0500x; 1.4020x over previous
import functools

import jax
import jax.numpy as jnp
from jax import lax
from jax.experimental import pallas as pl
from jax.experimental.pallas import tpu as pltpu
from jax.experimental.pallas import tpu_sc as plsc

N_NODES = 100000
NUM_FEATURES = 128
DIM = 16
NUM_CLASSES = 16
N_REL = 8
N_REL_EDGES = 64

NC, NS = 2, 16
E_PAD = 1638400
N_GROUPS = E_PAD // 128
GROUPS_PER_TILE = N_GROUPS // (NC * NS)
GROUPS_PER_CHUNK = 8
CHUNKS = GROUPS_PER_TILE // GROUPS_PER_CHUNK
N_PAD = 102400
TILE_ROWS = N_PAD // NS


def _rel_body(rel_x_ref, attr_ref, src_ref, dst_ref, nnw_ref, nnb_ref, out_ref):
    rel_x = rel_x_ref[...]
    attr = attr_ref[...]
    iota = lax.broadcasted_iota(jnp.int32, (N_REL_EDGES, N_REL), 1)
    oh_src = (src_ref[...] == iota).astype(jnp.float32)
    oh_dst = (dst_ref[...] == iota).astype(jnp.float32)
    ones = jnp.ones((N_REL_EDGES, 1), jnp.float32)
    cnt = lax.dot_general(oh_dst, ones, (((0,), (0,)), ((), ())))
    inv = 1.0 / jnp.maximum(cnt, 1.0)

    def conv(h):
        msgs = jnp.dot(oh_src, h) * attr
        summed = lax.dot_general(oh_dst, msgs, (((0,), (0,)), ((), ())))
        return jax.nn.relu(summed * inv + h)

    r = conv(conv(rel_x))
    out_ref[...] = jnp.dot(r, nnw_ref[...]) + nnb_ref[...]


def _rel_call(rel_x, rel_edge_attr, rel_edge_index, nn_W, nn_b):
    return pl.pallas_call(
        _rel_body,
        out_shape=jax.ShapeDtypeStruct((N_REL, DIM), jnp.float32),
    )(rel_x, rel_edge_attr[:, None],
      rel_edge_index[0][:, None], rel_edge_index[1][:, None],
      nn_W, nn_b[None, :])


def _gidx_body(src_ref, typ_ref, out_ref):
    out_ref[...] = src_ref[...] * N_REL + typ_ref[...]


def _gidx_call(src2d, typ2d):
    blk = (1600, 128)
    return pl.pallas_call(
        _gidx_body,
        out_shape=jax.ShapeDtypeStruct((N_GROUPS, 128), jnp.int32),
        grid=(N_GROUPS // blk[0],),
        in_specs=[pl.BlockSpec(blk, lambda i: (i, 0)),
                  pl.BlockSpec(blk, lambda i: (i, 0))],
        out_specs=pl.BlockSpec(blk, lambda i: (i, 0)),
    )(src2d, typ2d)


def _embed_body(x_ref, fc1_ref, w_ref, out_ref):
    h0 = jnp.dot(x_ref[...], fc1_ref[...],
                 preferred_element_type=jnp.float32)
    w = w_ref[...]
    out_ref[...] = jnp.concatenate(
        [h0 * w[t][None, :] for t in range(N_REL)], axis=1)


def _embed_call(x, fc1, w_rel):
    B = 1000
    return pl.pallas_call(
        _embed_body,
        out_shape=jax.ShapeDtypeStruct((N_PAD, N_REL * DIM), jnp.float32),
        grid=(N_NODES // B,),
        in_specs=[pl.BlockSpec((B, NUM_FEATURES), lambda i: (i, 0)),
                  pl.BlockSpec((NUM_FEATURES, DIM), lambda i: (0, 0)),
                  pl.BlockSpec((N_REL, DIM), lambda i: (0, 0))],
        out_specs=pl.BlockSpec((B, N_REL * DIM), lambda i: (i, 0)),
    )(x, fc1, w_rel)


EDGES_PER_TILE = E_PAD // (NC * NS)
EPC = 512
SC_CHUNKS = EDGES_PER_TILE // EPC


def _make_edge_pass(with_counts):
    mesh = plsc.VectorSubcoreMesh(
        core_axis_name="c", subcore_axis_name="s",
        num_cores=NC, num_subcores=NS)
    out_type = [jax.ShapeDtypeStruct((NC, N_PAD, DIM), jnp.float32)]
    if with_counts:
        out_type.append(jax.ShapeDtypeStruct((NC, 1, N_PAD), jnp.float32))
    scratch = [
        pltpu.VMEM_SHARED((N_PAD, DIM), jnp.float32),
        pltpu.VMEM_SHARED((N_PAD,), jnp.float32),
        pltpu.VMEM((EPC,), jnp.int32),
        pltpu.VMEM((EPC,), jnp.int32),
        pltpu.VMEM((EPC,), jnp.int32),
        pltpu.VMEM((EPC,), jnp.int32),
        pltpu.VMEM((EPC,), jnp.int32),
        pltpu.VMEM((EPC,), jnp.int32),
        pltpu.VMEM((EPC, DIM), jnp.float32),
        pltpu.VMEM((EPC, DIM), jnp.float32),
        pltpu.VMEM((EPC,), jnp.float32),
    ] + [pltpu.SemaphoreType.DMA] * 8

    def body(hall_ref, g_ref, d_ref, z2_ref, z1_ref, *rest):
        if with_counts:
            acc_out, cnt_out = rest[0], rest[1]
            rest = rest[2:]
        else:
            acc_out = rest[0]
            cnt_out = None
            rest = rest[1:]
        (acc_sh, cnt_sh, gb0, gb1, db0, db1, db2, db3, rw0, rw1, ones_v,
         isem0, isem1, gsem0, gsem1, ssem0, ssem1, csem0, csem1) = rest
        gbufs, dbufs, rows = [gb0, gb1], [db0, db1, db2, db3], [rw0, rw1]
        isems, gsems, ssems, csems = ([isem0, isem1], [gsem0, gsem1],
                                      [ssem0, ssem1], [csem0, csem1])

        cid = lax.axis_index("c")
        sid = lax.axis_index("s")
        off = sid * TILE_ROWS

        pltpu.sync_copy(z2_ref, acc_sh.at[pl.ds(off, TILE_ROWS)])
        if with_counts:
            pltpu.sync_copy(z1_ref, cnt_sh.at[pl.ds(off, TILE_ROWS)])
            for k in range(EPC // 16):
                ones_v[pl.ds(k * 16, 16)] = jnp.full((16,), 1.0, jnp.float32)
        plsc.subcore_barrier()

        ebase = (cid * NS + sid) * EDGES_PER_TILE

        def fire_idx(chunk, slot, dslot):
            base = ebase + chunk * EPC
            pltpu.async_copy(g_ref.at[pl.ds(base, EPC)], gbufs[slot],
                             isems[slot])
            pltpu.async_copy(d_ref.at[pl.ds(base, EPC)], dbufs[dslot],
                             isems[slot])

        def drain_idx(slot, dslot):
            pltpu.make_async_copy(g_ref.at[pl.ds(ebase, EPC)], gbufs[slot],
                                  isems[slot]).wait()
            pltpu.make_async_copy(d_ref.at[pl.ds(ebase, EPC)], dbufs[dslot],
                                  isems[slot]).wait()

        def fire_scat(slot, dslot):
            pltpu.async_copy(rows[slot], acc_sh.at[dbufs[dslot]], ssems[slot],
                             add=True)
            if with_counts:
                pltpu.async_copy(ones_v, cnt_sh.at[dbufs[dslot]], csems[slot],
                                 add=True)

        def drain_scat(slot, dslot):
            pltpu.make_async_copy(rows[slot], acc_sh.at[dbufs[dslot]],
                                  ssems[slot]).wait()
            if with_counts:
                pltpu.make_async_copy(ones_v, cnt_sh.at[dbufs[dslot]],
                                      csems[slot]).wait()

        def drain_gather(slot):
            pltpu.make_async_copy(hall_ref.at[gbufs[slot]], rows[slot],
                                  gsems[slot]).wait()

        fire_idx(0, 0, 0)

        def quad(k, carry):
            for q in range(4):
                s = q % 2
                j = 4 * k + q

                def step_drain_scat(s=s, q=q):
                    drain_scat(s, (q + 2) % 4)
                if q < 2:
                    pl.when(k > 0)(step_drain_scat)
                else:
                    step_drain_scat()

                drain_idx(s, q)
                pltpu.async_copy(hall_ref.at[gbufs[s]], rows[s], gsems[s])

                def step_tail(s=s, q=q):
                    drain_gather(1 - s)
                    fire_scat(1 - s, (q + 3) % 4)
                if q == 0:
                    pl.when(k > 0)(step_tail)
                else:
                    step_tail()
                fire_idx(jnp.minimum(j + 1, SC_CHUNKS - 1), 1 - s,
                         (q + 1) % 4)
            return carry

        lax.fori_loop(0, SC_CHUNKS // 4, quad, 0)

        drain_gather(1)
        fire_scat(1, 3)
        drain_scat(0, 2)
        drain_scat(1, 3)
        drain_idx(0, 0)
        plsc.subcore_barrier()

        pltpu.sync_copy(acc_sh.at[pl.ds(off, TILE_ROWS)],
                        acc_out.at[cid, pl.ds(off, TILE_ROWS)])
        if with_counts:
            pltpu.sync_copy(cnt_sh.at[pl.ds(off, TILE_ROWS)],
                            cnt_out.at[cid, 0, pl.ds(off, TILE_ROWS)])

    return pl.kernel(body, out_type=out_type, mesh=mesh, scratch_types=scratch,
                     compiler_params=pltpu.CompilerParams(
                         use_tc_tiling_on_sc=False))


_edge_pass_counts = _make_edge_pass(True)
_edge_pass_plain = _make_edge_pass(False)


def _epi1_body(acc_ref, cnt_ref, b_ref, w_ref, out_ref):
    i = pl.program_id(0)
    B = acc_ref.shape[1]
    a = acc_ref[0] + acc_ref[1]
    cnt = jnp.maximum(cnt_ref[0, 0, pl.ds(i * B, B)]
                      + cnt_ref[1, 0, pl.ds(i * B, B)], 1.0)
    h1 = jnp.tanh(a / cnt[:, None] + b_ref[...])
    w = w_ref[...]
    out_ref[...] = jnp.concatenate(
        [h1 * w[t][None, :] for t in range(N_REL)], axis=1)


def _epi1_call(acc, cnt3, conv1_b, w_rel):
    B = 1024
    return pl.pallas_call(
        _epi1_body,
        out_shape=jax.ShapeDtypeStruct((N_PAD, N_REL * DIM), jnp.float32),
        grid=(N_PAD // B,),
        in_specs=[pl.BlockSpec((NC, B, DIM), lambda i: (0, i, 0)),
                  pl.BlockSpec((NC, 1, N_PAD), lambda i: (0, 0, 0)),
                  pl.BlockSpec((1, DIM), lambda i: (0, 0)),
                  pl.BlockSpec((N_REL, DIM), lambda i: (0, 0))],
        out_specs=pl.BlockSpec((B, N_REL * DIM), lambda i: (i, 0)),
    )(acc, cnt3, conv1_b[None, :], w_rel)


def _final_body(acc_ref, cnt_ref, b_ref, w2_ref, b2_ref, out_ref):
    i = pl.program_id(0)
    B = acc_ref.shape[1]
    a = acc_ref[0] + acc_ref[1]
    cnt = jnp.maximum(cnt_ref[0, 0, pl.ds(i * B, B)]
                      + cnt_ref[1, 0, pl.ds(i * B, B)], 1.0)
    h2 = jnp.tanh(a / cnt[:, None] + b_ref[...])
    out_ref[...] = jnp.dot(h2, w2_ref[...],
                           preferred_element_type=jnp.float32) + b2_ref[...]


def _final_call(acc, cnt3, conv2_b, fc2_W, fc2_b):
    B = 1024
    return pl.pallas_call(
        _final_body,
        out_shape=jax.ShapeDtypeStruct((N_PAD, NUM_CLASSES), jnp.float32),
        grid=(N_PAD // B,),
        in_specs=[pl.BlockSpec((NC, B, DIM), lambda i: (0, i, 0)),
                  pl.BlockSpec((NC, 1, N_PAD), lambda i: (0, 0, 0)),
                  pl.BlockSpec((1, DIM), lambda i: (0, 0)),
                  pl.BlockSpec((DIM, NUM_CLASSES), lambda i: (0, 0)),
                  pl.BlockSpec((1, NUM_CLASSES), lambda i: (0, 0))],
        out_specs=pl.BlockSpec((B, NUM_CLASSES), lambda i: (i, 0)),
    )(acc, cnt3, conv2_b[None, :], fc2_W, fc2_b[None, :])


def kernel(x, rel_x, rel_edge_attr, fc1, nn_W, nn_b, conv1_b, conv2_b,
           fc2_W, fc2_b, edge_index, edge_type, rel_edge_index):
    E = edge_index.shape[1]
    pad = E_PAD - E
    src = jnp.concatenate([edge_index[0], jnp.zeros((pad,), jnp.int32)])
    dst = jnp.concatenate([edge_index[1],
                           jnp.full((pad,), N_NODES, jnp.int32)])
    typ = jnp.concatenate([edge_type, jnp.zeros((pad,), jnp.int32)])
    src2d = src.reshape(N_GROUPS, 128)
    typ2d = typ.reshape(N_GROUPS, 128)

    z2 = jnp.zeros((TILE_ROWS, DIM), jnp.float32)
    z1 = jnp.zeros((TILE_ROWS,), jnp.float32)

    w_rel = _rel_call(rel_x, rel_edge_attr, rel_edge_index, nn_W, nn_b)
    g1d = _gidx_call(src2d, typ2d).reshape(E_PAD)
    hall1 = _embed_call(x, fc1, w_rel).reshape(N_PAD * N_REL, DIM)

    acc1, cnt3 = _edge_pass_counts(hall1, g1d, dst, z2, z1)
    hall2 = _epi1_call(acc1, cnt3, conv1_b, w_rel).reshape(N_PAD * N_REL, DIM)

    acc2 = _edge_pass_plain(hall2, g1d, dst, z2, z1)
    if isinstance(acc2, (list, tuple)):
        acc2 = acc2[0]
    return _final_call(acc2, cnt3, conv2_b, fc2_W, fc2_b)[:N_NODES]


# speedup vs baseline: 20.4589x; 1.2747x over previous
import functools

import jax
import jax.numpy as jnp
from jax import lax
from jax.experimental import pallas as pl
from jax.experimental.pallas import tpu as pltpu
from jax.experimental.pallas import tpu_sc as plsc

N_NODES = 100000
NUM_FEATURES = 128
DIM = 16
NUM_CLASSES = 16
N_REL = 8
N_REL_EDGES = 64

NC, NS = 2, 16
E_PAD = 1638400
N_GROUPS = E_PAD // 128
GROUPS_PER_TILE = N_GROUPS // (NC * NS)
GROUPS_PER_CHUNK = 8
CHUNKS = GROUPS_PER_TILE // GROUPS_PER_CHUNK
N_PAD = 102400
TILE_ROWS = N_PAD // NS


def _rel_body(rel_x_ref, attr_ref, src_ref, dst_ref, nnw_ref, nnb_ref, out_ref):
    rel_x = rel_x_ref[...]
    attr = attr_ref[...]
    iota = lax.broadcasted_iota(jnp.int32, (N_REL_EDGES, N_REL), 1)
    oh_src = (src_ref[...] == iota).astype(jnp.float32)
    oh_dst = (dst_ref[...] == iota).astype(jnp.float32)
    ones = jnp.ones((N_REL_EDGES, 1), jnp.float32)
    cnt = lax.dot_general(oh_dst, ones, (((0,), (0,)), ((), ())))
    inv = 1.0 / jnp.maximum(cnt, 1.0)

    def conv(h):
        msgs = jnp.dot(oh_src, h) * attr
        summed = lax.dot_general(oh_dst, msgs, (((0,), (0,)), ((), ())))
        return jax.nn.relu(summed * inv + h)

    r = conv(conv(rel_x))
    out_ref[...] = jnp.dot(r, nnw_ref[...]) + nnb_ref[...]


def _rel_call(rel_x, rel_edge_attr, rel_edge_index, nn_W, nn_b):
    return pl.pallas_call(
        _rel_body,
        out_shape=jax.ShapeDtypeStruct((N_REL, DIM), jnp.float32),
    )(rel_x, rel_edge_attr[:, None],
      rel_edge_index[0][:, None], rel_edge_index[1][:, None],
      nn_W, nn_b[None, :])


def _gidx_body(src_ref, typ_ref, out_ref):
    out_ref[...] = src_ref[...] * N_REL + typ_ref[...]


def _gidx_call(src2d, typ2d):
    blk = (1600, 128)
    return pl.pallas_call(
        _gidx_body,
        out_shape=jax.ShapeDtypeStruct((N_GROUPS, 128), jnp.int32),
        grid=(N_GROUPS // blk[0],),
        in_specs=[pl.BlockSpec(blk, lambda i: (i, 0)),
                  pl.BlockSpec(blk, lambda i: (i, 0))],
        out_specs=pl.BlockSpec(blk, lambda i: (i, 0)),
    )(src2d, typ2d)


def _embed_body(x_ref, fc1_ref, w_ref, out_ref):
    h0 = jnp.dot(x_ref[...], fc1_ref[...],
                 preferred_element_type=jnp.float32)
    w = w_ref[...]
    out_ref[...] = jnp.concatenate(
        [h0 * w[t][None, :] for t in range(N_REL)], axis=1)


def _embed_call(x, fc1, w_rel):
    B = 1000
    return pl.pallas_call(
        _embed_body,
        out_shape=jax.ShapeDtypeStruct((N_PAD, N_REL * DIM), jnp.float32),
        grid=(N_NODES // B,),
        in_specs=[pl.BlockSpec((B, NUM_FEATURES), lambda i: (i, 0)),
                  pl.BlockSpec((NUM_FEATURES, DIM), lambda i: (0, 0)),
                  pl.BlockSpec((N_REL, DIM), lambda i: (0, 0))],
        out_specs=pl.BlockSpec((B, N_REL * DIM), lambda i: (i, 0)),
    )(x, fc1, w_rel)


EDGES_PER_TILE = E_PAD // (NC * NS)
EPC = 512
SC_CHUNKS = EDGES_PER_TILE // EPC


def _make_edge_pass(with_counts):
    mesh = plsc.VectorSubcoreMesh(
        core_axis_name="c", subcore_axis_name="s",
        num_cores=NC, num_subcores=NS)
    out_type = [jax.ShapeDtypeStruct((NC, N_PAD, DIM), jnp.float32)]
    if with_counts:
        out_type.append(jax.ShapeDtypeStruct((NC, 1, N_PAD), jnp.float32))
    scratch = [
        pltpu.VMEM_SHARED((N_PAD, DIM), jnp.float32),
        pltpu.VMEM_SHARED((N_PAD,), jnp.float32),
        pltpu.VMEM((EPC,), jnp.int32),
        pltpu.VMEM((EPC,), jnp.int32),
        pltpu.VMEM((EPC,), jnp.int32),
        pltpu.VMEM((EPC,), jnp.int32),
        pltpu.VMEM((EPC,), jnp.int32),
        pltpu.VMEM((EPC,), jnp.int32),
        pltpu.VMEM((EPC, DIM), jnp.float32),
        pltpu.VMEM((EPC, DIM), jnp.float32),
        pltpu.VMEM((EPC,), jnp.float32),
    ] + [pltpu.SemaphoreType.DMA] * 8

    def body(hall_ref, g_ref, d_ref, z2_ref, z1_ref, *rest):
        if with_counts:
            acc_out, cnt_out = rest[0], rest[1]
            rest = rest[2:]
        else:
            acc_out = rest[0]
            cnt_out = None
            rest = rest[1:]
        (acc_sh, cnt_sh, gb0, gb1, db0, db1, db2, db3, rw0, rw1, ones_v,
         isem0, isem1, gsem0, gsem1, ssem0, ssem1, csem0, csem1) = rest
        gbufs, dbufs, rows = [gb0, gb1], [db0, db1, db2, db3], [rw0, rw1]
        isems, gsems, ssems, csems = ([isem0, isem1], [gsem0, gsem1],
                                      [ssem0, ssem1], [csem0, csem1])

        cid = lax.axis_index("c")
        sid = lax.axis_index("s")
        off = sid * TILE_ROWS

        pltpu.sync_copy(z2_ref, acc_sh.at[pl.ds(off, TILE_ROWS)])
        if with_counts:
            pltpu.sync_copy(z1_ref, cnt_sh.at[pl.ds(off, TILE_ROWS)])
            for k in range(EPC // 16):
                ones_v[pl.ds(k * 16, 16)] = jnp.full((16,), 1.0, jnp.float32)
        plsc.subcore_barrier()

        ebase = (cid * NS + sid) * EDGES_PER_TILE

        def fire_idx(chunk, slot, dslot):
            base = ebase + chunk * EPC
            pltpu.async_copy(g_ref.at[pl.ds(base, EPC)], gbufs[slot],
                             isems[slot])
            pltpu.async_copy(d_ref.at[pl.ds(base, EPC)], dbufs[dslot],
                             isems[slot])

        def drain_idx(slot, dslot):
            pltpu.make_async_copy(g_ref.at[pl.ds(ebase, EPC)], gbufs[slot],
                                  isems[slot]).wait()
            pltpu.make_async_copy(d_ref.at[pl.ds(ebase, EPC)], dbufs[dslot],
                                  isems[slot]).wait()

        def fire_scat(slot, dslot):
            pltpu.async_copy(rows[slot], acc_sh.at[dbufs[dslot]], ssems[slot],
                             add=True)
            if with_counts:
                pltpu.async_copy(ones_v, cnt_sh.at[dbufs[dslot]], csems[slot],
                                 add=True)

        def drain_scat(slot, dslot):
            pltpu.make_async_copy(rows[slot], acc_sh.at[dbufs[dslot]],
                                  ssems[slot]).wait()
            if with_counts:
                pltpu.make_async_copy(ones_v, cnt_sh.at[dbufs[dslot]],
                                      csems[slot]).wait()

        def drain_gather(slot):
            pltpu.make_async_copy(hall_ref.at[gbufs[slot]], rows[slot],
                                  gsems[slot]).wait()

        fire_idx(0, 0, 0)

        def quad(k, carry):
            for q in range(4):
                s = q % 2
                j = 4 * k + q

                def step_drain_scat(s=s, q=q):
                    drain_scat(s, (q + 2) % 4)
                if q < 2:
                    pl.when(k > 0)(step_drain_scat)
                else:
                    step_drain_scat()

                drain_idx(s, q)
                pltpu.async_copy(hall_ref.at[gbufs[s]], rows[s], gsems[s])

                def step_tail(s=s, q=q):
                    drain_gather(1 - s)
                    fire_scat(1 - s, (q + 3) % 4)
                if q == 0:
                    pl.when(k > 0)(step_tail)
                else:
                    step_tail()
                fire_idx(jnp.minimum(j + 1, SC_CHUNKS - 1), 1 - s,
                         (q + 1) % 4)
            return carry

        lax.fori_loop(0, SC_CHUNKS // 4, quad, 0)

        drain_gather(1)
        fire_scat(1, 3)
        drain_scat(0, 2)
        drain_scat(1, 3)
        drain_idx(0, 0)
        plsc.subcore_barrier()

        pltpu.sync_copy(acc_sh.at[pl.ds(off, TILE_ROWS)],
                        acc_out.at[cid, pl.ds(off, TILE_ROWS)])
        if with_counts:
            pltpu.sync_copy(cnt_sh.at[pl.ds(off, TILE_ROWS)],
                            cnt_out.at[cid, 0, pl.ds(off, TILE_ROWS)])

    return pl.kernel(body, out_type=out_type, mesh=mesh, scratch_types=scratch,
                     compiler_params=pltpu.CompilerParams(
                         use_tc_tiling_on_sc=False))


_edge_pass_counts = _make_edge_pass(True)
_edge_pass_plain = _make_edge_pass(False)


NP8 = N_PAD // 8


def _expand_mat():
    ii = lax.broadcasted_iota(jnp.int32, (8, 128), 0)
    jj = lax.broadcasted_iota(jnp.int32, (8, 128), 1)
    return (jj // DIM == ii).astype(jnp.float32)


def _epi1_body(acc_ref, cnt_ref, b_ref, out_ref):
    a = acc_ref[0] + acc_ref[1]
    c2 = cnt_ref[0] + cnt_ref[1]
    cp = lax.dot_general(c2, _expand_mat(), (((1,), (0,)), ((), ())))
    b128 = jnp.concatenate([b_ref[...]] * 8, axis=1)
    out_ref[...] = jnp.tanh(a / jnp.maximum(cp, 1.0) + b128)


def _epi1_call(accp, cnt8, conv1_b):
    BP = 1280
    return pl.pallas_call(
        _epi1_body,
        out_shape=jax.ShapeDtypeStruct((NP8, 128), jnp.float32),
        grid=(NP8 // BP,),
        in_specs=[pl.BlockSpec((NC, BP, 128), lambda i: (0, i, 0)),
                  pl.BlockSpec((NC, BP, 8), lambda i: (0, i, 0)),
                  pl.BlockSpec((1, DIM), lambda i: (0, 0))],
        out_specs=pl.BlockSpec((BP, 128), lambda i: (i, 0)),
    )(accp, cnt8, conv1_b[None, :])


TB_NODES = N_PAD // (NC * NS)
TB_CB = 200


def _table_body(h_ref, w_ref, out_ref, inb, outb, wv):
    cid = lax.axis_index("c")
    sid = lax.axis_index("s")
    nbase = (cid * NS + sid) * TB_NODES
    pltpu.sync_copy(w_ref, wv)
    wrows = [wv[pl.ds(t * DIM, DIM)] for t in range(N_REL)]

    def chunk(ch, carry):
        base = nbase + ch * TB_CB
        pltpu.sync_copy(h_ref.at[pl.ds(base * DIM, TB_CB * DIM)], inb)

        def node(i, carry2):
            v = inb[pl.ds(i * DIM, DIM)]
            for t in range(N_REL):
                outb[pl.ds(i * 128 + t * DIM, DIM)] = v * wrows[t]
            return carry2

        lax.fori_loop(0, TB_CB, node, 0)
        pltpu.sync_copy(outb, out_ref.at[pl.ds(base * 128, TB_CB * 128)])
        return carry

    lax.fori_loop(0, TB_NODES // TB_CB, chunk, 0)


def _make_table():
    mesh = plsc.VectorSubcoreMesh(
        core_axis_name="c", subcore_axis_name="s",
        num_cores=NC, num_subcores=NS)
    scratch = [
        pltpu.VMEM((TB_CB * DIM,), jnp.float32),
        pltpu.VMEM((TB_CB * 128,), jnp.float32),
        pltpu.VMEM((N_REL * DIM,), jnp.float32),
    ]
    return pl.kernel(
        _table_body,
        out_type=jax.ShapeDtypeStruct((N_PAD * N_REL * DIM,), jnp.float32),
        mesh=mesh, scratch_types=scratch,
        compiler_params=pltpu.CompilerParams(use_tc_tiling_on_sc=False))


_table_call = _make_table()


def _final_body(acc_ref, cnt_ref, b_ref, w2_ref, b2_ref, out_ref):
    a = acc_ref[0] + acc_ref[1]
    c2 = cnt_ref[0] + cnt_ref[1]
    cp = lax.dot_general(c2, _expand_mat(), (((1,), (0,)), ((), ())))
    b128 = jnp.concatenate([b_ref[...]] * 8, axis=1)
    h2 = jnp.tanh(a / jnp.maximum(cp, 1.0) + b128)
    wrow = jnp.concatenate([w2_ref[...]] * 8, axis=1)
    wt = jnp.concatenate([wrow] * 8, axis=0)
    ii = lax.broadcasted_iota(jnp.int32, (128, 128), 0)
    jj = lax.broadcasted_iota(jnp.int32, (128, 128), 1)
    bd = wt * (ii // DIM == jj // NUM_CLASSES).astype(jnp.float32)
    b2128 = jnp.concatenate([b2_ref[...]] * 8, axis=1)
    out_ref[...] = jnp.dot(h2, bd,
                           preferred_element_type=jnp.float32) + b2128


def _final_call(accp, cnt8, conv2_b, fc2_W, fc2_b):
    BP = 1280
    return pl.pallas_call(
        _final_body,
        out_shape=jax.ShapeDtypeStruct((NP8, 128), jnp.float32),
        grid=(NP8 // BP,),
        in_specs=[pl.BlockSpec((NC, BP, 128), lambda i: (0, i, 0)),
                  pl.BlockSpec((NC, BP, 8), lambda i: (0, i, 0)),
                  pl.BlockSpec((1, DIM), lambda i: (0, 0)),
                  pl.BlockSpec((DIM, NUM_CLASSES), lambda i: (0, 0)),
                  pl.BlockSpec((1, NUM_CLASSES), lambda i: (0, 0))],
        out_specs=pl.BlockSpec((BP, 128), lambda i: (i, 0)),
    )(accp, cnt8, conv2_b[None, :], fc2_W, fc2_b[None, :])


def kernel(x, rel_x, rel_edge_attr, fc1, nn_W, nn_b, conv1_b, conv2_b,
           fc2_W, fc2_b, edge_index, edge_type, rel_edge_index):
    E = edge_index.shape[1]
    pad = E_PAD - E
    src = jnp.concatenate([edge_index[0], jnp.zeros((pad,), jnp.int32)])
    dst = jnp.concatenate([edge_index[1],
                           jnp.full((pad,), N_NODES, jnp.int32)])
    typ = jnp.concatenate([edge_type, jnp.zeros((pad,), jnp.int32)])
    src2d = src.reshape(N_GROUPS, 128)
    typ2d = typ.reshape(N_GROUPS, 128)

    z2 = jnp.zeros((TILE_ROWS, DIM), jnp.float32)
    z1 = jnp.zeros((TILE_ROWS,), jnp.float32)

    w_rel = _rel_call(rel_x, rel_edge_attr, rel_edge_index, nn_W, nn_b)
    g1d = _gidx_call(src2d, typ2d).reshape(E_PAD)
    hall1 = _embed_call(x, fc1, w_rel).reshape(N_PAD * N_REL, DIM)

    acc1, cnt3 = _edge_pass_counts(hall1, g1d, dst, z2, z1)
    accp1 = acc1.reshape(NC, NP8, 128)
    cnt8 = cnt3.reshape(NC, NP8, 8)
    h1p = _epi1_call(accp1, cnt8, conv1_b)
    hall2 = _table_call(h1p.reshape(N_PAD * DIM),
                        w_rel.reshape(N_REL * DIM)).reshape(
        N_PAD * N_REL, DIM)

    acc2 = _edge_pass_plain(hall2, g1d, dst, z2, z1)
    if isinstance(acc2, (list, tuple)):
        acc2 = acc2[0]
    accp2 = acc2.reshape(NC, NP8, 128)
    out = _final_call(accp2, cnt8, conv2_b, fc2_W, fc2_b)
    return out.reshape(N_PAD, DIM)[:N_NODES]


# speedup vs baseline: 20.7476x; 1.0141x over previous
import functools

import jax
import jax.numpy as jnp
from jax import lax
from jax.experimental import pallas as pl
from jax.experimental.pallas import tpu as pltpu
from jax.experimental.pallas import tpu_sc as plsc

N_NODES = 100000
NUM_FEATURES = 128
DIM = 16
NUM_CLASSES = 16
N_REL = 8
N_REL_EDGES = 64

NC, NS = 2, 16
E_PAD = 1638400
N_GROUPS = E_PAD // 128
GROUPS_PER_TILE = N_GROUPS // (NC * NS)
GROUPS_PER_CHUNK = 8
CHUNKS = GROUPS_PER_TILE // GROUPS_PER_CHUNK
N_PAD = 102400
TILE_ROWS = N_PAD // NS


def _rel_body(rel_x_ref, attr_ref, src_ref, dst_ref, nnw_ref, nnb_ref, out_ref):
    rel_x = rel_x_ref[...]
    attr = attr_ref[...]
    iota = lax.broadcasted_iota(jnp.int32, (N_REL_EDGES, N_REL), 1)
    oh_src = (src_ref[...] == iota).astype(jnp.float32)
    oh_dst = (dst_ref[...] == iota).astype(jnp.float32)
    ones = jnp.ones((N_REL_EDGES, 1), jnp.float32)
    cnt = lax.dot_general(oh_dst, ones, (((0,), (0,)), ((), ())))
    inv = 1.0 / jnp.maximum(cnt, 1.0)

    def conv(h):
        msgs = jnp.dot(oh_src, h) * attr
        summed = lax.dot_general(oh_dst, msgs, (((0,), (0,)), ((), ())))
        return jax.nn.relu(summed * inv + h)

    r = conv(conv(rel_x))
    out_ref[...] = jnp.dot(r, nnw_ref[...]) + nnb_ref[...]


def _rel_call(rel_x, rel_edge_attr, rel_edge_index, nn_W, nn_b):
    return pl.pallas_call(
        _rel_body,
        out_shape=jax.ShapeDtypeStruct((N_REL, DIM), jnp.float32),
    )(rel_x, rel_edge_attr[:, None],
      rel_edge_index[0][:, None], rel_edge_index[1][:, None],
      nn_W, nn_b[None, :])


def _gidx_body(src_ref, typ_ref, out_ref):
    out_ref[...] = src_ref[...] * N_REL + typ_ref[...]


def _gidx_call(src2d, typ2d):
    blk = (1600, 128)
    return pl.pallas_call(
        _gidx_body,
        out_shape=jax.ShapeDtypeStruct((N_GROUPS, 128), jnp.int32),
        grid=(N_GROUPS // blk[0],),
        in_specs=[pl.BlockSpec(blk, lambda i: (i, 0)),
                  pl.BlockSpec(blk, lambda i: (i, 0))],
        out_specs=pl.BlockSpec(blk, lambda i: (i, 0)),
    )(src2d, typ2d)


def _embed_body(x_ref, fc1_ref, w_ref, out_ref):
    h0 = jnp.dot(x_ref[...], fc1_ref[...],
                 preferred_element_type=jnp.float32)
    w = w_ref[...]
    out_ref[...] = jnp.concatenate(
        [h0 * w[t][None, :] for t in range(N_REL)], axis=1)


def _embed_call(x, fc1, w_rel):
    B = 2000
    return pl.pallas_call(
        _embed_body,
        out_shape=jax.ShapeDtypeStruct((N_PAD, N_REL * DIM), jnp.float32),
        grid=(N_NODES // B,),
        in_specs=[pl.BlockSpec((B, NUM_FEATURES), lambda i: (i, 0)),
                  pl.BlockSpec((NUM_FEATURES, DIM), lambda i: (0, 0)),
                  pl.BlockSpec((N_REL, DIM), lambda i: (0, 0))],
        out_specs=pl.BlockSpec((B, N_REL * DIM), lambda i: (i, 0)),
    )(x, fc1, w_rel)


EDGES_PER_TILE = E_PAD // (NC * NS)
EPC = 512
SC_CHUNKS = EDGES_PER_TILE // EPC


def _make_edge_pass(with_counts):
    mesh = plsc.VectorSubcoreMesh(
        core_axis_name="c", subcore_axis_name="s",
        num_cores=NC, num_subcores=NS)
    out_type = [jax.ShapeDtypeStruct((NC, N_PAD, DIM), jnp.float32)]
    if with_counts:
        out_type.append(jax.ShapeDtypeStruct((NC, 1, N_PAD), jnp.float32))
    scratch = [
        pltpu.VMEM_SHARED((N_PAD, DIM), jnp.float32),
        pltpu.VMEM_SHARED((N_PAD,), jnp.float32),
        pltpu.VMEM((EPC,), jnp.int32),
        pltpu.VMEM((EPC,), jnp.int32),
        pltpu.VMEM((EPC,), jnp.int32),
        pltpu.VMEM((EPC,), jnp.int32),
        pltpu.VMEM((EPC,), jnp.int32),
        pltpu.VMEM((EPC,), jnp.int32),
        pltpu.VMEM((EPC, DIM), jnp.float32),
        pltpu.VMEM((EPC, DIM), jnp.float32),
        pltpu.VMEM((EPC,), jnp.float32),
    ] + [pltpu.SemaphoreType.DMA] * 8

    def body(hall_ref, g_ref, d_ref, z2_ref, z1_ref, *rest):
        if with_counts:
            acc_out, cnt_out = rest[0], rest[1]
            rest = rest[2:]
        else:
            acc_out = rest[0]
            cnt_out = None
            rest = rest[1:]
        (acc_sh, cnt_sh, gb0, gb1, db0, db1, db2, db3, rw0, rw1, ones_v,
         isem0, isem1, gsem0, gsem1, ssem0, ssem1, csem0, csem1) = rest
        gbufs, dbufs, rows = [gb0, gb1], [db0, db1, db2, db3], [rw0, rw1]
        isems, gsems, ssems, csems = ([isem0, isem1], [gsem0, gsem1],
                                      [ssem0, ssem1], [csem0, csem1])

        cid = lax.axis_index("c")
        sid = lax.axis_index("s")
        off = sid * TILE_ROWS

        pltpu.sync_copy(z2_ref, acc_sh.at[pl.ds(off, TILE_ROWS)])
        if with_counts:
            pltpu.sync_copy(z1_ref, cnt_sh.at[pl.ds(off, TILE_ROWS)])
            for k in range(EPC // 16):
                ones_v[pl.ds(k * 16, 16)] = jnp.full((16,), 1.0, jnp.float32)
        plsc.subcore_barrier()

        ebase = (cid * NS + sid) * EDGES_PER_TILE

        def fire_idx(chunk, slot, dslot):
            base = ebase + chunk * EPC
            pltpu.async_copy(g_ref.at[pl.ds(base, EPC)], gbufs[slot],
                             isems[slot])
            pltpu.async_copy(d_ref.at[pl.ds(base, EPC)], dbufs[dslot],
                             isems[slot])

        def drain_idx(slot, dslot):
            pltpu.make_async_copy(g_ref.at[pl.ds(ebase, EPC)], gbufs[slot],
                                  isems[slot]).wait()
            pltpu.make_async_copy(d_ref.at[pl.ds(ebase, EPC)], dbufs[dslot],
                                  isems[slot]).wait()

        def fire_scat(slot, dslot):
            pltpu.async_copy(rows[slot], acc_sh.at[dbufs[dslot]], ssems[slot],
                             add=True)
            if with_counts:
                pltpu.async_copy(ones_v, cnt_sh.at[dbufs[dslot]], csems[slot],
                                 add=True)

        def drain_scat(slot, dslot):
            pltpu.make_async_copy(rows[slot], acc_sh.at[dbufs[dslot]],
                                  ssems[slot]).wait()
            if with_counts:
                pltpu.make_async_copy(ones_v, cnt_sh.at[dbufs[dslot]],
                                      csems[slot]).wait()

        def drain_gather(slot):
            pltpu.make_async_copy(hall_ref.at[gbufs[slot]], rows[slot],
                                  gsems[slot]).wait()

        fire_idx(0, 0, 0)

        def quad(k, carry):
            for q in range(4):
                s = q % 2
                j = 4 * k + q

                def step_drain_scat(s=s, q=q):
                    drain_scat(s, (q + 2) % 4)
                if q < 2:
                    pl.when(k > 0)(step_drain_scat)
                else:
                    step_drain_scat()

                drain_idx(s, q)
                pltpu.async_copy(hall_ref.at[gbufs[s]], rows[s], gsems[s])

                def step_tail(s=s, q=q):
                    drain_gather(1 - s)
                    fire_scat(1 - s, (q + 3) % 4)
                if q == 0:
                    pl.when(k > 0)(step_tail)
                else:
                    step_tail()
                fire_idx(jnp.minimum(j + 1, SC_CHUNKS - 1), 1 - s,
                         (q + 1) % 4)
            return carry

        lax.fori_loop(0, SC_CHUNKS // 4, quad, 0)

        drain_gather(1)
        fire_scat(1, 3)
        drain_scat(0, 2)
        drain_scat(1, 3)
        drain_idx(0, 0)
        plsc.subcore_barrier()

        pltpu.sync_copy(acc_sh.at[pl.ds(off, TILE_ROWS)],
                        acc_out.at[cid, pl.ds(off, TILE_ROWS)])
        if with_counts:
            pltpu.sync_copy(cnt_sh.at[pl.ds(off, TILE_ROWS)],
                            cnt_out.at[cid, 0, pl.ds(off, TILE_ROWS)])

    return pl.kernel(body, out_type=out_type, mesh=mesh, scratch_types=scratch,
                     compiler_params=pltpu.CompilerParams(
                         use_tc_tiling_on_sc=False))


_edge_pass_counts = _make_edge_pass(True)
_edge_pass_plain = _make_edge_pass(False)


NP8 = N_PAD // 8


def _expand_mat():
    ii = lax.broadcasted_iota(jnp.int32, (8, 128), 0)
    jj = lax.broadcasted_iota(jnp.int32, (8, 128), 1)
    return (jj // DIM == ii).astype(jnp.float32)


def _epi1_body(acc_ref, cnt_ref, b_ref, out_ref):
    a = acc_ref[0] + acc_ref[1]
    c2 = cnt_ref[0] + cnt_ref[1]
    cp = lax.dot_general(c2, _expand_mat(), (((1,), (0,)), ((), ())))
    b128 = jnp.concatenate([b_ref[...]] * 8, axis=1)
    out_ref[...] = jnp.tanh(a / jnp.maximum(cp, 1.0) + b128)


def _epi1_call(accp, cnt8, conv1_b):
    BP = 1280
    return pl.pallas_call(
        _epi1_body,
        out_shape=jax.ShapeDtypeStruct((NP8, 128), jnp.float32),
        grid=(NP8 // BP,),
        in_specs=[pl.BlockSpec((NC, BP, 128), lambda i: (0, i, 0)),
                  pl.BlockSpec((NC, BP, 8), lambda i: (0, i, 0)),
                  pl.BlockSpec((1, DIM), lambda i: (0, 0))],
        out_specs=pl.BlockSpec((BP, 128), lambda i: (i, 0)),
    )(accp, cnt8, conv1_b[None, :])


TB_NODES = N_PAD // (NC * NS)
TB_CB = 200


def _table_body(h_ref, w_ref, out_ref, inb, outb, wv):
    cid = lax.axis_index("c")
    sid = lax.axis_index("s")
    nbase = (cid * NS + sid) * TB_NODES
    pltpu.sync_copy(w_ref, wv)
    wrows = [wv[pl.ds(t * DIM, DIM)] for t in range(N_REL)]

    def chunk(ch, carry):
        base = nbase + ch * TB_CB
        pltpu.sync_copy(h_ref.at[pl.ds(base * DIM, TB_CB * DIM)], inb)

        def node(i2, carry2):
            for u in range(2):
                i = i2 * 2 + u
                v = inb[pl.ds(i * DIM, DIM)]
                for t in range(N_REL):
                    outb[pl.ds(i * 128 + t * DIM, DIM)] = v * wrows[t]
            return carry2

        lax.fori_loop(0, TB_CB // 2, node, 0)
        pltpu.sync_copy(outb, out_ref.at[pl.ds(base * 128, TB_CB * 128)])
        return carry

    lax.fori_loop(0, TB_NODES // TB_CB, chunk, 0)


def _make_table():
    mesh = plsc.VectorSubcoreMesh(
        core_axis_name="c", subcore_axis_name="s",
        num_cores=NC, num_subcores=NS)
    scratch = [
        pltpu.VMEM((TB_CB * DIM,), jnp.float32),
        pltpu.VMEM((TB_CB * 128,), jnp.float32),
        pltpu.VMEM((N_REL * DIM,), jnp.float32),
    ]
    return pl.kernel(
        _table_body,
        out_type=jax.ShapeDtypeStruct((N_PAD * N_REL * DIM,), jnp.float32),
        mesh=mesh, scratch_types=scratch,
        compiler_params=pltpu.CompilerParams(use_tc_tiling_on_sc=False))


_table_call = _make_table()


def _final_body(acc_ref, cnt_ref, b_ref, w2_ref, b2_ref, out_ref):
    a = acc_ref[0] + acc_ref[1]
    c2 = cnt_ref[0] + cnt_ref[1]
    cp = lax.dot_general(c2, _expand_mat(), (((1,), (0,)), ((), ())))
    b128 = jnp.concatenate([b_ref[...]] * 8, axis=1)
    h2 = jnp.tanh(a / jnp.maximum(cp, 1.0) + b128)
    wrow = jnp.concatenate([w2_ref[...]] * 8, axis=1)
    wt = jnp.concatenate([wrow] * 8, axis=0)
    ii = lax.broadcasted_iota(jnp.int32, (128, 128), 0)
    jj = lax.broadcasted_iota(jnp.int32, (128, 128), 1)
    bd = wt * (ii // DIM == jj // NUM_CLASSES).astype(jnp.float32)
    b2128 = jnp.concatenate([b2_ref[...]] * 8, axis=1)
    out_ref[...] = jnp.dot(h2, bd,
                           preferred_element_type=jnp.float32) + b2128


def _final_call(accp, cnt8, conv2_b, fc2_W, fc2_b):
    BP = 1280
    return pl.pallas_call(
        _final_body,
        out_shape=jax.ShapeDtypeStruct((NP8, 128), jnp.float32),
        grid=(NP8 // BP,),
        in_specs=[pl.BlockSpec((NC, BP, 128), lambda i: (0, i, 0)),
                  pl.BlockSpec((NC, BP, 8), lambda i: (0, i, 0)),
                  pl.BlockSpec((1, DIM), lambda i: (0, 0)),
                  pl.BlockSpec((DIM, NUM_CLASSES), lambda i: (0, 0)),
                  pl.BlockSpec((1, NUM_CLASSES), lambda i: (0, 0))],
        out_specs=pl.BlockSpec((BP, 128), lambda i: (i, 0)),
    )(accp, cnt8, conv2_b[None, :], fc2_W, fc2_b[None, :])


def kernel(x, rel_x, rel_edge_attr, fc1, nn_W, nn_b, conv1_b, conv2_b,
           fc2_W, fc2_b, edge_index, edge_type, rel_edge_index):
    E = edge_index.shape[1]
    pad = E_PAD - E
    src = jnp.concatenate([edge_index[0], jnp.zeros((pad,), jnp.int32)])
    dst = jnp.concatenate([edge_index[1],
                           jnp.full((pad,), N_NODES, jnp.int32)])
    typ = jnp.concatenate([edge_type, jnp.zeros((pad,), jnp.int32)])
    src2d = src.reshape(N_GROUPS, 128)
    typ2d = typ.reshape(N_GROUPS, 128)

    z2 = jnp.zeros((TILE_ROWS, DIM), jnp.float32)
    z1 = jnp.zeros((TILE_ROWS,), jnp.float32)

    w_rel = _rel_call(rel_x, rel_edge_attr, rel_edge_index, nn_W, nn_b)
    g1d = _gidx_call(src2d, typ2d).reshape(E_PAD)
    hall1 = _embed_call(x, fc1, w_rel).reshape(N_PAD * N_REL, DIM)

    acc1, cnt3 = _edge_pass_counts(hall1, g1d, dst, z2, z1)
    accp1 = acc1.reshape(NC, NP8, 128)
    cnt8 = cnt3.reshape(NC, NP8, 8)
    h1p = _epi1_call(accp1, cnt8, conv1_b)
    hall2 = _table_call(h1p.reshape(N_PAD * DIM),
                        w_rel.reshape(N_REL * DIM)).reshape(
        N_PAD * N_REL, DIM)

    acc2 = _edge_pass_plain(hall2, g1d, dst, z2, z1)
    if isinstance(acc2, (list, tuple)):
        acc2 = acc2[0]
    accp2 = acc2.reshape(NC, NP8, 128)
    out = _final_call(accp2, cnt8, conv2_b, fc2_W, fc2_b)
    return out.reshape(N_PAD, DIM)[:N_NODES]
